# Initial kernel scaffold; baseline (speedup 1.0000x reference)
import jax
import jax.numpy as jnp
from jax.experimental import pallas as pl


def kernel(x, edge_index, edge_attr, W1, b1, W2, b2, U1, c1, U2, c2, gamma, beta):
    raise NotImplementedError("write your pallas kernel here")


# speedup vs baseline: 1.5059x; 1.5059x over previous
import functools

import jax
import jax.numpy as jnp
from jax import lax
from jax.experimental import pallas as pl
from jax.experimental.pallas import tpu as pltpu
from jax.experimental.pallas import tpu_sc as plsc

N_NODES = 10000
N_EDGES = 320000
D = 128
ED = 16

NC = 2
NS = 16
NW = NC * NS
WIN = 128

E_PAD = 327680
CHUNKS_PER_W = E_PAD // (NW * WIN)
N_AGG = 10112
ROWS_PER_SUB = N_AGG // NS

EBLK = 1024
NBLK = 400


def _mish(v):
    return v * jnp.tanh(jax.nn.softplus(v))


def _sc_gather(x, src_idx, dst_idx):
    mesh = plsc.VectorSubcoreMesh(core_axis_name="c", subcore_axis_name="s")
    n_win = E_PAD // WIN

    @functools.partial(
        pl.kernel,
        out_type=(
            jax.ShapeDtypeStruct((E_PAD, D), jnp.float32),
            jax.ShapeDtypeStruct((E_PAD, D), jnp.float32),
        ),
        mesh=mesh,
    )
    def gather_kernel(x_hbm, si_hbm, di_hbm, xs_hbm, xd_hbm):
        def body(si_v, di_v, xs_v, xd_v):
            pltpu.sync_copy(x_hbm.at[si_v.at[0]], xs_v)
            pltpu.sync_copy(x_hbm.at[di_v.at[0]], xd_v)

        pltpu.emit_pipeline(
            body,
            grid=(n_win,),
            in_specs=[
                pl.BlockSpec((1, WIN), lambda i: (0, i)),
                pl.BlockSpec((1, WIN), lambda i: (0, i)),
            ],
            out_specs=[
                pl.BlockSpec((WIN, D), lambda i: (i, 0)),
                pl.BlockSpec((WIN, D), lambda i: (i, 0)),
            ],
            core_axis_name=("c", "s"),
            dimension_semantics=(pltpu.PARALLEL,),
        )(si_hbm, di_hbm, xs_hbm, xd_hbm)

    return gather_kernel(x, src_idx.reshape(1, E_PAD), dst_idx.reshape(1, E_PAD))


def _edge_mlp_body(xs_ref, xd_ref, ea_ref, w1s_ref, w1d_ref, w1e_ref, b1_ref,
                   w2_ref, b2_ref, m_ref):
    h = (
        jnp.dot(xs_ref[...], w1s_ref[...], preferred_element_type=jnp.float32)
        + jnp.dot(xd_ref[...], w1d_ref[...], preferred_element_type=jnp.float32)
        + jnp.dot(ea_ref[...], w1e_ref[...], preferred_element_type=jnp.float32)
        + b1_ref[...]
    )
    h = _mish(h)
    h = jnp.dot(h, w2_ref[...], preferred_element_type=jnp.float32) + b2_ref[...]
    m_ref[...] = _mish(h)


def _edge_mlp(xs, xd, ea_pad, W1, b1, W2, b2):
    grid = (E_PAD // EBLK,)
    return pl.pallas_call(
        _edge_mlp_body,
        grid=grid,
        in_specs=[
            pl.BlockSpec((EBLK, D), lambda i: (i, 0)),
            pl.BlockSpec((EBLK, D), lambda i: (i, 0)),
            pl.BlockSpec((EBLK, ED), lambda i: (i, 0)),
            pl.BlockSpec((D, D), lambda i: (0, 0)),
            pl.BlockSpec((D, D), lambda i: (0, 0)),
            pl.BlockSpec((ED, D), lambda i: (0, 0)),
            pl.BlockSpec((1, D), lambda i: (0, 0)),
            pl.BlockSpec((D, D), lambda i: (0, 0)),
            pl.BlockSpec((1, D), lambda i: (0, 0)),
        ],
        out_specs=pl.BlockSpec((EBLK, D), lambda i: (i, 0)),
        out_shape=jax.ShapeDtypeStruct((E_PAD, D), jnp.float32),
    )(xs, xd, ea_pad, W1[:D], W1[D:2 * D], W1[2 * D:], b1.reshape(1, D),
      W2, b2.reshape(1, D))


def _sc_scatter_add(m, dst_idx, zeros_init):
    mesh = plsc.VectorSubcoreMesh(core_axis_name="c", subcore_axis_name="s")

    @functools.partial(
        pl.kernel,
        out_type=jax.ShapeDtypeStruct((NC, N_AGG, D), jnp.float32),
        mesh=mesh,
        scratch_types=[
            pltpu.VMEM((CHUNKS_PER_W, WIN), jnp.int32),
            pltpu.VMEM((WIN, D), jnp.float32),
            pltpu.VMEM_SHARED((N_AGG, D), jnp.float32),
        ],
    )
    def scatter_kernel(m_hbm, di_hbm, z_hbm, out_hbm, idx_v, mbuf_v, agg_sh):
        cid = lax.axis_index("c")
        sid = lax.axis_index("s")
        wid = sid * NC + cid

        pltpu.sync_copy(
            z_hbm.at[pl.ds(sid * ROWS_PER_SUB, ROWS_PER_SUB)],
            agg_sh.at[pl.ds(sid * ROWS_PER_SUB, ROWS_PER_SUB)],
        )

        pltpu.sync_copy(di_hbm.at[pl.ds(wid * CHUNKS_PER_W, CHUNKS_PER_W)], idx_v)
        plsc.subcore_barrier()

        base = wid * CHUNKS_PER_W * WIN

        @pl.loop(0, CHUNKS_PER_W)
        def _(j):
            pltpu.sync_copy(m_hbm.at[pl.ds(base + j * WIN, WIN)], mbuf_v)
            pltpu.sync_copy(mbuf_v, agg_sh.at[idx_v.at[j]], add=True)

        plsc.subcore_barrier()
        pltpu.sync_copy(
            agg_sh.at[pl.ds(sid * ROWS_PER_SUB, ROWS_PER_SUB)],
            out_hbm.at[cid].at[pl.ds(sid * ROWS_PER_SUB, ROWS_PER_SUB)],
        )

    return scatter_kernel(m, dst_idx.reshape(NW * CHUNKS_PER_W, WIN), zeros_init)


def _node_mlp_body(x_ref, agg_ref, u1x_ref, u1a_ref, c1_ref, u2_ref, c2_ref,
                   g_ref, bt_ref, o_ref):
    x = x_ref[...]
    agg = agg_ref[0] + agg_ref[1]
    u = (
        jnp.dot(x, u1x_ref[...], preferred_element_type=jnp.float32)
        + jnp.dot(agg, u1a_ref[...], preferred_element_type=jnp.float32)
        + c1_ref[...]
    )
    o = jnp.dot(_mish(u), u2_ref[...], preferred_element_type=jnp.float32) + c2_ref[...]
    r = x + o
    mu = jnp.mean(r, axis=-1, keepdims=True)
    var = jnp.mean((r - mu) ** 2, axis=-1, keepdims=True)
    o_ref[...] = (r - mu) * jax.lax.rsqrt(var + 1e-5) * g_ref[...] + bt_ref[...]


def _node_mlp(x, partials, U1, c1, U2, c2, gamma, beta):
    grid = (N_NODES // NBLK,)
    return pl.pallas_call(
        _node_mlp_body,
        grid=grid,
        in_specs=[
            pl.BlockSpec((NBLK, D), lambda i: (i, 0)),
            pl.BlockSpec((NC, NBLK, D), lambda i: (0, i, 0)),
            pl.BlockSpec((D, D), lambda i: (0, 0)),
            pl.BlockSpec((D, D), lambda i: (0, 0)),
            pl.BlockSpec((1, D), lambda i: (0, 0)),
            pl.BlockSpec((D, D), lambda i: (0, 0)),
            pl.BlockSpec((1, D), lambda i: (0, 0)),
            pl.BlockSpec((1, D), lambda i: (0, 0)),
            pl.BlockSpec((1, D), lambda i: (0, 0)),
        ],
        out_specs=pl.BlockSpec((NBLK, D), lambda i: (i, 0)),
        out_shape=jax.ShapeDtypeStruct((N_NODES, D), jnp.float32),
    )(x, partials, U1[:D], U1[D:], c1.reshape(1, D), U2, c2.reshape(1, D),
      gamma.reshape(1, D), beta.reshape(1, D))


def kernel(x, edge_index, edge_attr, W1, b1, W2, b2, U1, c1, U2, c2, gamma, beta):
    pad = E_PAD - N_EDGES
    src = jnp.concatenate([edge_index[0], jnp.zeros((pad,), jnp.int32)])
    dst = jnp.concatenate([edge_index[1], jnp.full((pad,), N_NODES, jnp.int32)])
    ea = jnp.concatenate([edge_attr, jnp.zeros((pad, ED), jnp.float32)], axis=0)

    xs, xd = _sc_gather(x, src, dst)
    m = _edge_mlp(xs, xd, ea, W1, b1, W2, b2)
    zeros_init = jnp.zeros((N_AGG, D), jnp.float32)
    partials = _sc_scatter_add(m, dst, zeros_init)
    return _node_mlp(x, partials, U1, c1, U2, c2, gamma, beta)


# speedup vs baseline: 1.8743x; 1.2447x over previous
import functools

import jax
import jax.numpy as jnp
from jax import lax
from jax.experimental import pallas as pl
from jax.experimental.pallas import tpu as pltpu
from jax.experimental.pallas import tpu_sc as plsc

N_NODES = 10000
N_EDGES = 320000
D = 128
DP = D // 2
ED = 16

NC = 2
NS = 16
NW = NC * NS
WIN = 128

E_PAD = 327680
CHUNKS_PER_W = E_PAD // (NW * WIN)
N_AGG = 10112
ROWS_PER_SUB = N_AGG // NS

EBLK = 1024
NBLK = 400


def _mish(v):
    return v * jnp.tanh(jax.nn.softplus(v))


NBUF = 4
WIN_PER_W = 2 * E_PAD // (NW * WIN)


def _sc_gather(x, idx_all):
    mesh = plsc.VectorSubcoreMesh(core_axis_name="c", subcore_axis_name="s")

    @functools.partial(
        pl.kernel,
        out_type=jax.ShapeDtypeStruct((2 * E_PAD, D), jnp.float32),
        mesh=mesh,
        scratch_types=[
            pltpu.VMEM((WIN_PER_W, WIN), jnp.int32),
            [pltpu.VMEM((WIN, D), jnp.float32) for _ in range(NBUF)],
            [pltpu.SemaphoreType.DMA for _ in range(NBUF)],
            [pltpu.SemaphoreType.DMA for _ in range(NBUF)],
        ],
    )
    def gather_kernel(x_hbm, i_hbm, g_hbm, ibuf, bufs, gsems, wsems):
        cid = lax.axis_index("c")
        sid = lax.axis_index("s")
        wid = sid * NC + cid
        row0 = wid * WIN_PER_W * WIN

        pltpu.sync_copy(i_hbm.at[pl.ds(wid * WIN_PER_W, WIN_PER_W)], ibuf)

        def gather_desc(g, b):
            return pltpu.make_async_copy(
                x_hbm.at[ibuf.at[g]], bufs[b], gsems[b])

        def wout_desc(g, b):
            return pltpu.make_async_copy(
                bufs[b], g_hbm.at[pl.ds(row0 + g * WIN, WIN)], wsems[b])

        for b in range(NBUF):
            gather_desc(b, b).start()

        @pl.loop(0, WIN_PER_W, step=NBUF)
        def _(j):
            for b in range(NBUF):
                g = j + b
                gather_desc(g, b).wait()
                wout_desc(g, b).start()
                b2 = (b + 2) % NBUF
                g2 = g + 2

                @pl.when(jnp.logical_and(g >= 2, g2 < WIN_PER_W))
                def _():
                    wout_desc(g - 2, b2).wait()
                    gather_desc(g2, b2).start()

        for b in range(NBUF):
            wout_desc(WIN_PER_W - NBUF + b, b).wait()

    return gather_kernel(x, idx_all.reshape(NW * WIN_PER_W, WIN))


def _edge_mlp_body(xs_ref, xd_ref, ea_ref, w1s_ref, w1d_ref, w1e_ref,
                   b1_ref, w2_ref, b2_ref, m_ref):
    bf = jnp.bfloat16
    h = (
        jnp.dot(xs_ref[...].astype(bf), w1s_ref[...], preferred_element_type=jnp.float32)
        + jnp.dot(xd_ref[...].astype(bf), w1d_ref[...], preferred_element_type=jnp.float32)
        + jnp.dot(ea_ref[...], w1e_ref[...], preferred_element_type=jnp.float32)
        + b1_ref[...]
    )
    h = _mish(h)
    h = jnp.dot(h.astype(jnp.bfloat16), w2_ref[...],
                preferred_element_type=jnp.float32) + b2_ref[...]
    m_ref[...] = _mish(h)


def _edge_mlp(g, ea_pad, W1, b1, W2, b2):
    bf = jnp.bfloat16
    W1s, W1d, W1e = W1[:D].astype(bf), W1[D:2 * D].astype(bf), W1[2 * D:]
    nblk = E_PAD // EBLK
    return pl.pallas_call(
        _edge_mlp_body,
        grid=(nblk,),
        in_specs=[
            pl.BlockSpec((EBLK, D), lambda i: (i, 0)),
            pl.BlockSpec((EBLK, D), lambda i, _n=nblk: (i + _n, 0)),
            pl.BlockSpec((EBLK, ED), lambda i: (i, 0)),
            pl.BlockSpec((D, D), lambda i: (0, 0)),
            pl.BlockSpec((D, D), lambda i: (0, 0)),
            pl.BlockSpec((ED, D), lambda i: (0, 0)),
            pl.BlockSpec((1, D), lambda i: (0, 0)),
            pl.BlockSpec((D, D), lambda i: (0, 0)),
            pl.BlockSpec((1, D), lambda i: (0, 0)),
        ],
        out_specs=pl.BlockSpec((EBLK, D), lambda i: (i, 0)),
        out_shape=jax.ShapeDtypeStruct((E_PAD, D), jnp.float32),
    )(g, g, ea_pad, W1s, W1d, W1e, b1.reshape(1, D), W2.astype(bf),
      b2.reshape(1, D))


def _sc_scatter_add(m, dst_idx, zeros_init):
    mesh = plsc.VectorSubcoreMesh(core_axis_name="c", subcore_axis_name="s")

    @functools.partial(
        pl.kernel,
        out_type=jax.ShapeDtypeStruct((NC, N_AGG, D), jnp.float32),
        mesh=mesh,
        scratch_types=[
            pltpu.VMEM((CHUNKS_PER_W, WIN), jnp.int32),
            pltpu.VMEM((WIN, D), jnp.float32),
            pltpu.VMEM_SHARED((N_AGG, D), jnp.float32),
        ],
    )
    def scatter_kernel(m_hbm, di_hbm, z_hbm, out_hbm, idx_v, mbuf_v, agg_sh):
        cid = lax.axis_index("c")
        sid = lax.axis_index("s")
        wid = sid * NC + cid

        pltpu.sync_copy(
            z_hbm.at[pl.ds(sid * ROWS_PER_SUB, ROWS_PER_SUB)],
            agg_sh.at[pl.ds(sid * ROWS_PER_SUB, ROWS_PER_SUB)],
        )

        pltpu.sync_copy(di_hbm.at[pl.ds(wid * CHUNKS_PER_W, CHUNKS_PER_W)], idx_v)
        plsc.subcore_barrier()

        base = wid * CHUNKS_PER_W * WIN

        @pl.loop(0, CHUNKS_PER_W)
        def _(j):
            pltpu.sync_copy(m_hbm.at[pl.ds(base + j * WIN, WIN)], mbuf_v)
            pltpu.sync_copy(mbuf_v, agg_sh.at[idx_v.at[j]], add=True)

        plsc.subcore_barrier()
        pltpu.sync_copy(
            agg_sh.at[pl.ds(sid * ROWS_PER_SUB, ROWS_PER_SUB)],
            out_hbm.at[cid].at[pl.ds(sid * ROWS_PER_SUB, ROWS_PER_SUB)],
        )

    return scatter_kernel(m, dst_idx.reshape(NW * CHUNKS_PER_W, WIN), zeros_init)


def _node_mlp_body(x_ref, agg_ref, u1x_ref, u1a_ref, c1_ref, u2_ref, c2_ref,
                   g_ref, bt_ref, o_ref):
    x = x_ref[...]
    agg = agg_ref[0] + agg_ref[1]
    u = (
        jnp.dot(x, u1x_ref[...], preferred_element_type=jnp.float32)
        + jnp.dot(agg, u1a_ref[...], preferred_element_type=jnp.float32)
        + c1_ref[...]
    )
    o = jnp.dot(_mish(u), u2_ref[...], preferred_element_type=jnp.float32) + c2_ref[...]
    r = x + o
    mu = jnp.mean(r, axis=-1, keepdims=True)
    var = jnp.mean((r - mu) ** 2, axis=-1, keepdims=True)
    o_ref[...] = (r - mu) * jax.lax.rsqrt(var + 1e-5) * g_ref[...] + bt_ref[...]


def _node_mlp(x, partials, U1, c1, U2, c2, gamma, beta):
    grid = (N_NODES // NBLK,)
    return pl.pallas_call(
        _node_mlp_body,
        grid=grid,
        in_specs=[
            pl.BlockSpec((NBLK, D), lambda i: (i, 0)),
            pl.BlockSpec((NC, NBLK, D), lambda i: (0, i, 0)),
            pl.BlockSpec((D, D), lambda i: (0, 0)),
            pl.BlockSpec((D, D), lambda i: (0, 0)),
            pl.BlockSpec((1, D), lambda i: (0, 0)),
            pl.BlockSpec((D, D), lambda i: (0, 0)),
            pl.BlockSpec((1, D), lambda i: (0, 0)),
            pl.BlockSpec((1, D), lambda i: (0, 0)),
            pl.BlockSpec((1, D), lambda i: (0, 0)),
        ],
        out_specs=pl.BlockSpec((NBLK, D), lambda i: (i, 0)),
        out_shape=jax.ShapeDtypeStruct((N_NODES, D), jnp.float32),
    )(x, partials, U1[:D], U1[D:], c1.reshape(1, D), U2, c2.reshape(1, D),
      gamma.reshape(1, D), beta.reshape(1, D))


def kernel(x, edge_index, edge_attr, W1, b1, W2, b2, U1, c1, U2, c2, gamma, beta):
    pad = E_PAD - N_EDGES
    src = jnp.concatenate([edge_index[0], jnp.zeros((pad,), jnp.int32)])
    dst = jnp.concatenate([edge_index[1], jnp.full((pad,), N_NODES, jnp.int32)])
    ea = jnp.concatenate([edge_attr, jnp.zeros((pad, ED), jnp.float32)], axis=0)
    idx_all = jnp.concatenate([src, dst])

    g = _sc_gather(x, idx_all)
    m = _edge_mlp(g, ea, W1, b1, W2, b2)
    zeros_init = jnp.zeros((N_AGG, D), jnp.float32)
    partials = _sc_scatter_add(m, dst, zeros_init)
    return _node_mlp(x, partials, U1, c1, U2, c2, gamma, beta)


# speedup vs baseline: 2.4094x; 1.2855x over previous
import functools

import jax
import jax.numpy as jnp
from jax import lax
from jax.experimental import pallas as pl
from jax.experimental.pallas import tpu as pltpu
from jax.experimental.pallas import tpu_sc as plsc

N_NODES = 10000
N_EDGES = 320000
D = 128
ED = 16

NC = 2
NS = 16
NW = NC * NS
WIN = 128

K_CH = 4
E_PAD = 327680
E_CH = E_PAD // K_CH
GWIN_PER_W = 2 * E_CH // (NW * WIN)
SWIN_PER_W = E_CH // (NW * WIN)
N_AGG = 10112
ROWS_PER_SUB = N_AGG // NS

NBUF = 4
EBLK = 1024
NBLK = 400


def _mish(v):
    t = jnp.exp(jnp.minimum(v, 40.0))
    u = t * (t + 2.0)
    return v * (u / (u + 2.0))


def _sc_gather(x, idx_chunk):
    mesh = plsc.VectorSubcoreMesh(core_axis_name="c", subcore_axis_name="s")

    @functools.partial(
        pl.kernel,
        out_type=jax.ShapeDtypeStruct((2 * E_CH, D), jnp.float32),
        mesh=mesh,
        scratch_types=[
            pltpu.VMEM((GWIN_PER_W, WIN), jnp.int32),
            [pltpu.VMEM((WIN, D), jnp.float32) for _ in range(NBUF)],
            [pltpu.SemaphoreType.DMA for _ in range(NBUF)],
            [pltpu.SemaphoreType.DMA for _ in range(NBUF)],
        ],
    )
    def gather_kernel(x_hbm, i_hbm, g_hbm, ibuf, bufs, gsems, wsems):
        cid = lax.axis_index("c")
        sid = lax.axis_index("s")
        wid = sid * NC + cid
        row0 = wid * GWIN_PER_W * WIN

        pltpu.sync_copy(i_hbm.at[pl.ds(wid * GWIN_PER_W, GWIN_PER_W)], ibuf)

        def gather_desc(g, b):
            return pltpu.make_async_copy(
                x_hbm.at[ibuf.at[g]], bufs[b], gsems[b])

        def wout_desc(g, b):
            return pltpu.make_async_copy(
                bufs[b], g_hbm.at[pl.ds(row0 + g * WIN, WIN)], wsems[b])

        for b in range(NBUF):
            gather_desc(b, b).start()

        @pl.loop(0, GWIN_PER_W, step=NBUF)
        def _(j):
            for b in range(NBUF):
                g = j + b
                gather_desc(g, b).wait()
                wout_desc(g, b).start()
                b2 = (b + 2) % NBUF
                g2 = g + 2

                @pl.when(jnp.logical_and(g >= 2, g2 < GWIN_PER_W))
                def _():
                    wout_desc(g - 2, b2).wait()
                    gather_desc(g2, b2).start()

        for b in range(NBUF):
            wout_desc(GWIN_PER_W - NBUF + b, b).wait()

    return gather_kernel(x, idx_chunk.reshape(NW * GWIN_PER_W, WIN))


def _edge_mlp_body(xs_ref, xd_ref, ea_ref, w1s_ref, w1d_ref, w1e_ref,
                   b1_ref, w2_ref, b2_ref, m_ref):
    bf = jnp.bfloat16
    h = (
        jnp.dot(xs_ref[...].astype(bf), w1s_ref[...], preferred_element_type=jnp.float32)
        + jnp.dot(xd_ref[...].astype(bf), w1d_ref[...], preferred_element_type=jnp.float32)
        + jnp.dot(ea_ref[...], w1e_ref[...], preferred_element_type=jnp.float32)
        + b1_ref[...]
    )
    h = _mish(h)
    h = jnp.dot(h.astype(bf), w2_ref[...],
                preferred_element_type=jnp.float32) + b2_ref[...]
    m_ref[...] = _mish(h)


def _edge_mlp(g, ea_chunk, W1s, W1d, W1e, b1, W2, b2):
    nblk = E_CH // EBLK
    return pl.pallas_call(
        _edge_mlp_body,
        grid=(nblk,),
        in_specs=[
            pl.BlockSpec((EBLK, D), lambda i: (i, 0)),
            pl.BlockSpec((EBLK, D), lambda i, _n=nblk: (i + _n, 0)),
            pl.BlockSpec((EBLK, ED), lambda i: (i, 0)),
            pl.BlockSpec((D, D), lambda i: (0, 0)),
            pl.BlockSpec((D, D), lambda i: (0, 0)),
            pl.BlockSpec((ED, D), lambda i: (0, 0)),
            pl.BlockSpec((1, D), lambda i: (0, 0)),
            pl.BlockSpec((D, D), lambda i: (0, 0)),
            pl.BlockSpec((1, D), lambda i: (0, 0)),
        ],
        out_specs=pl.BlockSpec((EBLK, D), lambda i: (i, 0)),
        out_shape=jax.ShapeDtypeStruct((E_CH, D), jnp.float32),
    )(g, g, ea_chunk, W1s, W1d, W1e, b1, W2, b2)


def _sc_scatter_add(m, dst_chunk):
    mesh = plsc.VectorSubcoreMesh(core_axis_name="c", subcore_axis_name="s")

    @functools.partial(
        pl.kernel,
        out_type=jax.ShapeDtypeStruct((NC, N_AGG, D), jnp.float32),
        mesh=mesh,
        scratch_types=[
            pltpu.VMEM((SWIN_PER_W, WIN), jnp.int32),
            pltpu.VMEM((WIN, D), jnp.float32),
            pltpu.VMEM_SHARED((N_AGG, D), jnp.float32),
        ],
    )
    def scatter_kernel(m_hbm, di_hbm, out_hbm, idx_v, mbuf_v, agg_sh):
        cid = lax.axis_index("c")
        sid = lax.axis_index("s")
        wid = sid * NC + cid

        @pl.loop(0, WIN)
        def _(r):
            @pl.loop(0, D, step=16)
            def _(q):
                mbuf_v[r, pl.ds(q, 16)] = jnp.zeros((16,), jnp.float32)

        zrows = (WIN, WIN, WIN, WIN, ROWS_PER_SUB - 4 * WIN)
        off = sid * ROWS_PER_SUB
        for i, zr in enumerate(zrows):
            pltpu.sync_copy(
                mbuf_v.at[pl.ds(0, zr)],
                agg_sh.at[pl.ds(off + i * WIN, zr)],
            )

        pltpu.sync_copy(di_hbm.at[wid], idx_v)
        plsc.subcore_barrier()

        base = wid * SWIN_PER_W * WIN

        @pl.loop(0, SWIN_PER_W)
        def _(j):
            pltpu.sync_copy(m_hbm.at[pl.ds(base + j * WIN, WIN)], mbuf_v)
            pltpu.sync_copy(mbuf_v, agg_sh.at[idx_v.at[j]], add=True)

        plsc.subcore_barrier()
        pltpu.sync_copy(
            agg_sh.at[pl.ds(sid * ROWS_PER_SUB, ROWS_PER_SUB)],
            out_hbm.at[cid].at[pl.ds(sid * ROWS_PER_SUB, ROWS_PER_SUB)],
        )

    return scatter_kernel(m, dst_chunk.reshape(NW, SWIN_PER_W, WIN))


def _node_mlp_body(x_ref, p0_ref, p1_ref, p2_ref, p3_ref, u1x_ref, u1a_ref,
                   c1_ref, u2_ref, c2_ref, g_ref, bt_ref, o_ref):
    x = x_ref[...]
    agg = (
        (p0_ref[0] + p0_ref[1]) + (p1_ref[0] + p1_ref[1])
        + (p2_ref[0] + p2_ref[1]) + (p3_ref[0] + p3_ref[1])
    )
    u = (
        jnp.dot(x, u1x_ref[...], preferred_element_type=jnp.float32)
        + jnp.dot(agg, u1a_ref[...], preferred_element_type=jnp.float32)
        + c1_ref[...]
    )
    o = jnp.dot(_mish(u), u2_ref[...], preferred_element_type=jnp.float32) + c2_ref[...]
    r = x + o
    mu = jnp.mean(r, axis=-1, keepdims=True)
    var = jnp.mean((r - mu) ** 2, axis=-1, keepdims=True)
    o_ref[...] = (r - mu) * jax.lax.rsqrt(var + 1e-5) * g_ref[...] + bt_ref[...]


def _node_mlp(x, partials, U1, c1, U2, c2, gamma, beta):
    grid = (N_NODES // NBLK,)
    pspec = pl.BlockSpec((NC, NBLK, D), lambda i: (0, i, 0))
    wspec = pl.BlockSpec((D, D), lambda i: (0, 0))
    vspec = pl.BlockSpec((1, D), lambda i: (0, 0))
    return pl.pallas_call(
        _node_mlp_body,
        grid=grid,
        in_specs=[pl.BlockSpec((NBLK, D), lambda i: (i, 0)),
                  pspec, pspec, pspec, pspec,
                  wspec, wspec, vspec, wspec, vspec, vspec, vspec],
        out_specs=pl.BlockSpec((NBLK, D), lambda i: (i, 0)),
        out_shape=jax.ShapeDtypeStruct((N_NODES, D), jnp.float32),
    )(x, *partials, U1[:D], U1[D:], c1.reshape(1, D), U2, c2.reshape(1, D),
      gamma.reshape(1, D), beta.reshape(1, D))


def kernel(x, edge_index, edge_attr, W1, b1, W2, b2, U1, c1, U2, c2, gamma, beta):
    pad = E_PAD - N_EDGES
    src = jnp.concatenate([edge_index[0], jnp.zeros((pad,), jnp.int32)])
    dst = jnp.concatenate([edge_index[1], jnp.full((pad,), N_NODES, jnp.int32)])
    ea = jnp.concatenate([edge_attr, jnp.zeros((pad, ED), jnp.float32)], axis=0)

    bf = jnp.bfloat16
    W1s, W1d = W1[:D].astype(bf), W1[D:2 * D].astype(bf)
    W1e, b1r = W1[2 * D:], b1.reshape(1, D)
    W2b, b2r = W2.astype(bf), b2.reshape(1, D)

    partials = []
    for k in range(K_CH):
        sl = slice(k * E_CH, (k + 1) * E_CH)
        idx_k = jnp.concatenate([src[sl], dst[sl]])
        g_k = _sc_gather(x, idx_k)
        m_k = _edge_mlp(g_k, ea[sl], W1s, W1d, W1e, b1r, W2b, b2r)
        partials.append(_sc_scatter_add(m_k, dst[sl]))

    return _node_mlp(x, partials, U1, c1, U2, c2, gamma, beta)
